# baseline (device time: 97112 ns/iter reference)
import functools

import jax
import jax.numpy as jnp
from jax import lax
from jax.experimental import pallas as pl
from jax.experimental.pallas import tpu as pltpu

NZ = 4
NQ = 4
K = 16


def _xy_coords(p):
    px = p // 2
    return px, px ^ (p % 2)


def kernel(ids, E):
    v_local, d = E.shape
    t = ids.shape[0]
    tq = t // NQ
    sr = tq // NZ
    th2 = sr // 2

    def body(ids_ref, e_ref, out_ref, x_ref, gsem, acc_ref,
             rs_buf, srs, rrs, sag, rag,
             b_l0, b_r0, b_l1, b_r1, sb, rb):
        mx = lax.axis_index("x")
        my = lax.axis_index("y")
        mz = lax.axis_index("z")
        r = 2 * mx + (mx ^ my)
        rxr, ryr = _xy_coords(lax.rem(r + 1, NQ))
        rxl, ryl = _xy_coords(lax.rem(r + NQ - 1, NQ))
        right = (rxr, ryr, mz)
        left = (rxl, ryl, mz)
        o_l = lax.rem(r + NQ - 1, NQ)
        o_r = lax.rem(r + 1, NQ)
        o_d = lax.rem(r + 2, NQ)

        zpeers = [lax.rem(mz + k, NZ) for k in (1, 2, 3)]
        p1 = jnp.where(mz <= 1, 3, 0)
        p2 = jnp.where(mz <= 1, 2, jnp.where(mz == 2, 3, 1))
        p3 = jnp.where(mz == 1, 0, jnp.where(mz == 3, 2, 1))
        zfar = [p1, p2, p3]
        neighbors = tuple((mx, my, g) for g in zpeers) + (left, right)

        def seg_sl(g):
            return pl.ds(g * sr, sr)

        base = mz * v_local
        goff = r * tq

        def owned(tk):
            loc = ids_ref[goff + tk] - base
            return loc, (loc >= 0) & (loc < v_local)

        def gcp(tk):
            loc, _ = owned(tk)
            return pltpu.make_async_copy(
                e_ref.at[loc], x_ref.at[tk], gsem.at[lax.rem(tk, K)]
            )

        def gather_seg(g):
            lo = g * sr

            def lp(i, c):
                tk = lo + i
                tkm = lo + lax.max(i - K, 0)
                _, ow_prev = owned(tkm)

                @pl.when((i >= K) & ow_prev)
                def _():
                    gcp(tkm).wait()

                _, ow = owned(tk)

                @pl.when(ow)
                def _():
                    gcp(tk).start()

                return c

            lax.fori_loop(0, sr, lp, 0)

            def ep(i, c):
                tk = lo + sr - K + i
                _, ow = owned(tk)

                @pl.when(ow)
                def _():
                    gcp(tk).wait()

                return c

            lax.fori_loop(0, K, ep, 0)

        x_ref[...] = jnp.zeros_like(x_ref)
        barrier_sem = pltpu.get_barrier_semaphore()
        for nbr in neighbors:
            pl.semaphore_signal(
                barrier_sem, inc=1,
                device_id=nbr, device_id_type=pl.DeviceIdType.MESH,
            )
        gather_seg(zfar[0])
        pl.semaphore_wait(barrier_sem, len(neighbors))

        sends = []

        for k, g in enumerate(zfar):
            rdma = pltpu.make_async_remote_copy(
                src_ref=x_ref.at[seg_sl(g)],
                dst_ref=rs_buf.at[mz],
                send_sem=srs.at[k],
                recv_sem=rrs.at[mz],
                device_id=(mx, my, g),
                device_id_type=pl.DeviceIdType.MESH,
            )
            rdma.start()
            sends.append(rdma)
            gather_seg(zfar[k + 1] if k < 2 else mz)

        for g in zpeers:
            pltpu.make_async_remote_copy(
                src_ref=rs_buf.at[g], dst_ref=rs_buf.at[g],
                send_sem=srs.at[0], recv_sem=rrs.at[g],
                device_id=(mx, my, g),
                device_id_type=pl.DeviceIdType.MESH,
            ).wait_recv()
        own = seg_sl(mz)
        acc_ref[own, :] = (
            x_ref[own, :] + rs_buf[zpeers[0], :, :]
            + rs_buf[zpeers[1], :, :] + rs_buf[zpeers[2], :, :]
        )

        for k, g in enumerate(zfar):
            rdma = pltpu.make_async_remote_copy(
                src_ref=acc_ref.at[own],
                dst_ref=acc_ref.at[own],
                send_sem=sag.at[k],
                recv_sem=rag.at[mz],
                device_id=(mx, my, g),
                device_id_type=pl.DeviceIdType.MESH,
            )
            rdma.start()
            sends.append(rdma)

        def b_rdma(i, g, src, dst, dev):
            return pltpu.make_async_remote_copy(
                src_ref=src, dst_ref=dst, send_sem=sb.at[i, g],
                recv_sem=rb.at[i, g], device_id=dev,
                device_id_type=pl.DeviceIdType.MESH,
            )

        B = {}
        bsegs = [mz, p3, p2, p1]
        for n, g in enumerate(bsegs):
            sl = seg_sl(g)
            if n > 0:
                pltpu.make_async_remote_copy(
                    src_ref=acc_ref.at[sl], dst_ref=acc_ref.at[sl],
                    send_sem=sag.at[0], recv_sem=rag.at[g],
                    device_id=(mx, my, g),
                    device_id_type=pl.DeviceIdType.MESH,
                ).wait_recv()
            B[0, n] = b_rdma(0, g, acc_ref.at[sl], b_l0.at[sl], right)
            B[1, n] = b_rdma(1, g, acc_ref.at[sl], b_r0.at[sl], left)
            B[0, n].start()
            B[1, n].start()
            out_ref[pl.ds(r * tq + g * sr, sr), :] = acc_ref[sl, :]
        for n, g in enumerate(bsegs):
            B[0, n].wait_recv()
            B[2, n] = b_rdma(
                2, g, b_l0.at[pl.ds(g * sr, th2)],
                b_l1.at[pl.ds(g * th2, th2)], right)
            B[2, n].start()
            B[1, n].wait_recv()
            B[3, n] = b_rdma(
                3, g, b_r0.at[pl.ds(g * sr + th2, th2)],
                b_r1.at[pl.ds(g * th2, th2)], left)
            B[3, n].start()
            out_ref[pl.ds(o_l * tq + g * sr, sr), :] = \
                b_l0[pl.ds(g * sr, sr), :]
            out_ref[pl.ds(o_r * tq + g * sr, sr), :] = \
                b_r0[pl.ds(g * sr, sr), :]
        for n, g in enumerate(bsegs):
            B[2, n].wait_recv()
            B[3, n].wait_recv()
            out_ref[pl.ds(o_d * tq + g * sr, th2), :] = \
                b_l1[pl.ds(g * th2, th2), :]
            out_ref[pl.ds(o_d * tq + g * sr + th2, th2), :] = \
                b_r1[pl.ds(g * th2, th2), :]

        for rdma in sends + list(B.values()):
            rdma.wait_send()

        @functools.partial(
            pl.run_scoped, exit_sem=pltpu.SemaphoreType.REGULAR
        )
        def _(exit_sem):
            for nbr in neighbors:
                pl.semaphore_signal(
                    exit_sem, inc=1,
                    device_id=nbr, device_id_type=pl.DeviceIdType.MESH,
                )
            pl.semaphore_wait(exit_sem, len(neighbors))

    return pl.pallas_call(
        body,
        out_shape=jax.ShapeDtypeStruct((t, d), jnp.float32),
        in_specs=[
            pl.BlockSpec(memory_space=pltpu.SMEM),
            pl.BlockSpec(memory_space=pl.ANY),
        ],
        out_specs=pl.BlockSpec(memory_space=pltpu.VMEM),
        scratch_shapes=[
            pltpu.VMEM((tq, d), jnp.float32),
            pltpu.SemaphoreType.DMA((K,)),
            pltpu.VMEM((tq, d), jnp.float32),
            pltpu.VMEM((NZ, sr, d), jnp.float32),
            pltpu.SemaphoreType.DMA((NZ - 1,)),
            pltpu.SemaphoreType.DMA((NZ,)),
            pltpu.SemaphoreType.DMA((NZ - 1,)),
            pltpu.SemaphoreType.DMA((NZ,)),
            pltpu.VMEM((tq, d), jnp.float32),
            pltpu.VMEM((tq, d), jnp.float32),
            pltpu.VMEM((tq // 2, d), jnp.float32),
            pltpu.VMEM((tq // 2, d), jnp.float32),
            pltpu.SemaphoreType.DMA((4, NZ)),
            pltpu.SemaphoreType.DMA((4, NZ)),
        ],
        compiler_params=pltpu.CompilerParams(collective_id=0),
    )(ids, E)


# device time: 92184 ns/iter; 1.0535x vs baseline; 1.0535x over previous
import functools

import jax
import jax.numpy as jnp
from jax import lax
from jax.experimental import pallas as pl
from jax.experimental.pallas import tpu as pltpu

NZ = 4
NQ = 4
K = 16


def _xy_coords(p):
    px = p // 2
    return px, px ^ (p % 2)


def kernel(ids, E):
    v_local, d = E.shape
    t = ids.shape[0]
    tq = t // NQ
    sr = tq // NZ
    th2 = sr // 2

    def body(ids_ref, e_ref, out_ref, x_ref, gsem, acc_ref,
             rs_buf, srs, rrs, sag, rag,
             b_l0, b_r0, b_l1, b_r1, sb, rb):
        mx = lax.axis_index("x")
        my = lax.axis_index("y")
        mz = lax.axis_index("z")
        r = 2 * mx + (mx ^ my)
        rxr, ryr = _xy_coords(lax.rem(r + 1, NQ))
        rxl, ryl = _xy_coords(lax.rem(r + NQ - 1, NQ))
        right = (rxr, ryr, mz)
        left = (rxl, ryl, mz)
        o_l = lax.rem(r + NQ - 1, NQ)
        o_r = lax.rem(r + 1, NQ)
        o_d = lax.rem(r + 2, NQ)

        zpeers = [lax.rem(mz + k, NZ) for k in (1, 2, 3)]
        neighbors = tuple((mx, my, g) for g in zpeers) + (left, right)

        def seg_sl(g):
            return pl.ds(g * sr, sr)

        base = mz * v_local
        goff = r * tq

        def owned(tk):
            loc = ids_ref[goff + tk] - base
            return loc, (loc >= 0) & (loc < v_local)

        def gcp(tk):
            loc, _ = owned(tk)
            return pltpu.make_async_copy(
                e_ref.at[loc], x_ref.at[tk], gsem.at[lax.rem(tk, K)]
            )

        def gather_seg(g):
            lo = g * sr

            def lp(i, c):
                tk = lo + i
                tkm = lo + lax.max(i - K, 0)
                _, ow_prev = owned(tkm)

                @pl.when((i >= K) & ow_prev)
                def _():
                    gcp(tkm).wait()

                _, ow = owned(tk)

                @pl.when(ow)
                def _():
                    gcp(tk).start()

                return c

            lax.fori_loop(0, sr, lp, 0)

            def ep(i, c):
                tk = lo + sr - K + i
                _, ow = owned(tk)

                @pl.when(ow)
                def _():
                    gcp(tk).wait()

                return c

            lax.fori_loop(0, K, ep, 0)

        x_ref[...] = jnp.zeros_like(x_ref)
        barrier_sem = pltpu.get_barrier_semaphore()
        for nbr in neighbors:
            pl.semaphore_signal(
                barrier_sem, inc=1,
                device_id=nbr, device_id_type=pl.DeviceIdType.MESH,
            )
        gather_seg(zpeers[0])
        pl.semaphore_wait(barrier_sem, len(neighbors))

        sends = []

        for k, g in enumerate(zpeers):
            rdma = pltpu.make_async_remote_copy(
                src_ref=x_ref.at[seg_sl(g)],
                dst_ref=rs_buf.at[mz],
                send_sem=srs.at[k],
                recv_sem=rrs.at[mz],
                device_id=(mx, my, g),
                device_id_type=pl.DeviceIdType.MESH,
            )
            rdma.start()
            sends.append(rdma)
            gather_seg(zpeers[k + 1] if k < 2 else mz)

        for g in zpeers:
            pltpu.make_async_remote_copy(
                src_ref=rs_buf.at[g], dst_ref=rs_buf.at[g],
                send_sem=srs.at[0], recv_sem=rrs.at[g],
                device_id=(mx, my, g),
                device_id_type=pl.DeviceIdType.MESH,
            ).wait_recv()
        own = seg_sl(mz)
        acc_ref[own, :] = (
            x_ref[own, :] + rs_buf[zpeers[0], :, :]
            + rs_buf[zpeers[1], :, :] + rs_buf[zpeers[2], :, :]
        )

        for k, g in enumerate(zpeers):
            rdma = pltpu.make_async_remote_copy(
                src_ref=acc_ref.at[own],
                dst_ref=acc_ref.at[own],
                send_sem=sag.at[k],
                recv_sem=rag.at[mz],
                device_id=(mx, my, g),
                device_id_type=pl.DeviceIdType.MESH,
            )
            rdma.start()
            sends.append(rdma)

        def b_rdma(i, g, src, dst, dev):
            return pltpu.make_async_remote_copy(
                src_ref=src, dst_ref=dst, send_sem=sb.at[i, g],
                recv_sem=rb.at[i, g], device_id=dev,
                device_id_type=pl.DeviceIdType.MESH,
            )

        B = {}
        bsegs = [mz] + zpeers
        for n, g in enumerate(bsegs):
            sl = seg_sl(g)
            if n > 0:
                pltpu.make_async_remote_copy(
                    src_ref=acc_ref.at[sl], dst_ref=acc_ref.at[sl],
                    send_sem=sag.at[0], recv_sem=rag.at[g],
                    device_id=(mx, my, g),
                    device_id_type=pl.DeviceIdType.MESH,
                ).wait_recv()
            B[0, n] = b_rdma(0, g, acc_ref.at[sl], b_l0.at[sl], right)
            B[1, n] = b_rdma(1, g, acc_ref.at[sl], b_r0.at[sl], left)
            B[0, n].start()
            B[1, n].start()
            out_ref[pl.ds(r * tq + g * sr, sr), :] = acc_ref[sl, :]
        for n, g in enumerate(bsegs):
            B[0, n].wait_recv()
            B[1, n].wait_recv()
            B[2, n] = b_rdma(
                2, g, b_l0.at[pl.ds(g * sr, th2)],
                b_l1.at[pl.ds(g * th2, th2)], right)
            B[3, n] = b_rdma(
                3, g, b_r0.at[pl.ds(g * sr + th2, th2)],
                b_r1.at[pl.ds(g * th2, th2)], left)
            B[2, n].start()
            B[3, n].start()
            out_ref[pl.ds(o_l * tq + g * sr, sr), :] = \
                b_l0[pl.ds(g * sr, sr), :]
            out_ref[pl.ds(o_r * tq + g * sr, sr), :] = \
                b_r0[pl.ds(g * sr, sr), :]
        for n, g in enumerate(bsegs):
            B[2, n].wait_recv()
            B[3, n].wait_recv()
            out_ref[pl.ds(o_d * tq + g * sr, th2), :] = \
                b_l1[pl.ds(g * th2, th2), :]
            out_ref[pl.ds(o_d * tq + g * sr + th2, th2), :] = \
                b_r1[pl.ds(g * th2, th2), :]

        for rdma in sends + list(B.values()):
            rdma.wait_send()

        @functools.partial(
            pl.run_scoped, exit_sem=pltpu.SemaphoreType.REGULAR
        )
        def _(exit_sem):
            for nbr in neighbors:
                pl.semaphore_signal(
                    exit_sem, inc=1,
                    device_id=nbr, device_id_type=pl.DeviceIdType.MESH,
                )
            pl.semaphore_wait(exit_sem, len(neighbors))

    return pl.pallas_call(
        body,
        out_shape=jax.ShapeDtypeStruct((t, d), jnp.float32),
        in_specs=[
            pl.BlockSpec(memory_space=pltpu.SMEM),
            pl.BlockSpec(memory_space=pl.ANY),
        ],
        out_specs=pl.BlockSpec(memory_space=pltpu.VMEM),
        scratch_shapes=[
            pltpu.VMEM((tq, d), jnp.float32),
            pltpu.SemaphoreType.DMA((K,)),
            pltpu.VMEM((tq, d), jnp.float32),
            pltpu.VMEM((NZ, sr, d), jnp.float32),
            pltpu.SemaphoreType.DMA((NZ - 1,)),
            pltpu.SemaphoreType.DMA((NZ,)),
            pltpu.SemaphoreType.DMA((NZ - 1,)),
            pltpu.SemaphoreType.DMA((NZ,)),
            pltpu.VMEM((tq, d), jnp.float32),
            pltpu.VMEM((tq, d), jnp.float32),
            pltpu.VMEM((tq // 2, d), jnp.float32),
            pltpu.VMEM((tq // 2, d), jnp.float32),
            pltpu.SemaphoreType.DMA((4, NZ)),
            pltpu.SemaphoreType.DMA((4, NZ)),
        ],
        compiler_params=pltpu.CompilerParams(collective_id=0),
    )(ids, E)
